# baseline (speedup 1.0000x reference)
import functools

import jax
import jax.numpy as jnp
from jax import lax
from jax.experimental import pallas as pl
from jax.experimental.pallas import tpu as pltpu
from jax.experimental.pallas import tpu_sc as plsc

EMBED_DIM = 64
CHUNK = 512
LANES = 16
NUM_WORKERS = 32


def _emb_body(z, table, out3, table_v, idx_v, rows_v, sem_i, sem_w):
    n = z.shape[0]
    per_w = n // NUM_WORKERS
    n_chunks = per_w // CHUNK
    wid = lax.axis_index("s") * 2 + lax.axis_index("c")
    base = wid * per_w

    pltpu.sync_copy(table, table_v)
    pltpu.async_copy(z.at[pl.ds(base, CHUNK)], idx_v.at[0], sem_i.at[0])

    lanes_iota = lax.iota(jnp.int32, LANES)

    def pair(i, carry):
        for b in range(2):
            ci = 2 * i + b
            e0 = base + ci * CHUNK
            pltpu.make_async_copy(
                z.at[pl.ds(e0, CHUNK)], idx_v.at[b], sem_i.at[b]).wait()
            @pl.when(ci + 1 < n_chunks)
            def _():
                pltpu.async_copy(
                    z.at[pl.ds(e0 + CHUNK, CHUNK)], idx_v.at[1 - b],
                    sem_i.at[1 - b])
            @pl.when(ci >= 2)
            def _():
                pltpu.make_async_copy(
                    rows_v.at[b], out3.at[pl.ds(e0, CHUNK)],
                    sem_w.at[b]).wait()

            def group(g, carry2):
                idxv = idx_v[b, pl.ds(g * LANES, LANES)]
                rowv = g * LANES + lanes_iota
                for c in range(EMBED_DIM):
                    colv = jnp.full((LANES,), c, jnp.int32)
                    x = plsc.load_gather(table_v, [idxv, colv])
                    plsc.store_scatter(rows_v.at[b], [rowv, colv], x)
                return carry2

            lax.fori_loop(0, CHUNK // LANES, group, 0)

            pltpu.async_copy(rows_v.at[b], out3.at[pl.ds(e0, CHUNK)],
                             sem_w.at[b])
        return carry

    lax.fori_loop(0, n_chunks // 2, pair, 0)

    for b in range(2):
        e0 = base + (n_chunks - 2 + b) * CHUNK
        pltpu.make_async_copy(
            rows_v.at[b], out3.at[pl.ds(e0, CHUNK)], sem_w.at[b]).wait()


@jax.jit
def kernel(z, table):
    B, S = z.shape
    n = B * S
    z_flat = z.reshape(n).astype(jnp.int32)
    table = table.at[0].set(jnp.zeros((table.shape[1],), table.dtype))

    mesh = plsc.VectorSubcoreMesh(core_axis_name="c", subcore_axis_name="s")
    out3 = pl.kernel(
        _emb_body,
        mesh=mesh,
        out_type=jax.ShapeDtypeStruct((n, EMBED_DIM), jnp.float32),
        scratch_types=[
            pltpu.VMEM((100, EMBED_DIM), jnp.float32),
            pltpu.VMEM((2, CHUNK), jnp.int32),
            pltpu.VMEM((2, CHUNK, EMBED_DIM), jnp.float32),
            pltpu.SemaphoreType.DMA((2,)),
            pltpu.SemaphoreType.DMA((2,)),
        ],
        compiler_params=pltpu.CompilerParams(use_tc_tiling_on_sc=False,
                                             needs_layout_passes=False),
    )(z_flat, table)
    return out3.reshape(B, S, EMBED_DIM)


# speedup vs baseline: 3.8251x; 3.8251x over previous
import functools

import jax
import jax.numpy as jnp
from jax import lax
from jax.experimental import pallas as pl
from jax.experimental.pallas import tpu as pltpu
from jax.experimental.pallas import tpu_sc as plsc

EMBED_DIM = 64
CHUNK = 512
LANES = 16
NUM_WORKERS = 32


def _emb_body(z, table, out1, table_v, idx_v, rows_v, sem_i, sem_w):
    n = z.shape[0]
    per_w = n // NUM_WORKERS
    n_chunks = per_w // CHUNK
    wid = lax.axis_index("s") * 2 + lax.axis_index("c")
    base = wid * per_w

    pltpu.sync_copy(table, table_v)
    pltpu.async_copy(z.at[pl.ds(base, CHUNK)],
                     idx_v.at[pl.ds(0, CHUNK)], sem_i.at[0])

    iota64 = lax.iota(jnp.int32, LANES) * EMBED_DIM

    def pair(i, carry):
        for b in range(2):
            ci = 2 * i + b
            e0 = base + ci * CHUNK
            pltpu.make_async_copy(
                z.at[pl.ds(e0, CHUNK)],
                idx_v.at[pl.ds(b * CHUNK, CHUNK)], sem_i.at[b]).wait()
            @pl.when(ci + 1 < n_chunks)
            def _():
                pltpu.async_copy(
                    z.at[pl.ds(e0 + CHUNK, CHUNK)],
                    idx_v.at[pl.ds((1 - b) * CHUNK, CHUNK)],
                    sem_i.at[1 - b])
            @pl.when(ci >= 2)
            def _():
                pltpu.make_async_copy(
                    rows_v.at[pl.ds(b * CHUNK * EMBED_DIM,
                                    CHUNK * EMBED_DIM)],
                    out1.at[pl.ds(e0 * EMBED_DIM, CHUNK * EMBED_DIM)],
                    sem_w.at[b]).wait()


            @plsc.parallel_loop(0, CHUNK // LANES, step=1, unroll=1)
            def _grp(g):
                idxv = idx_v[pl.ds(b * CHUNK + g * LANES, LANES)]
                base_d = (b * CHUNK * EMBED_DIM) + (g << 10)
                for l in range(LANES):
                    a = idxv[l] << 6
                    d = base_d + (l << 6)
                    for q in range(EMBED_DIM // LANES):
                        rows_v[pl.ds(d + q * LANES, LANES)] = (
                            table_v[pl.ds(a + q * LANES, LANES)])

            pltpu.async_copy(
                rows_v.at[pl.ds(b * CHUNK * EMBED_DIM, CHUNK * EMBED_DIM)],
                out1.at[pl.ds(e0 * EMBED_DIM, CHUNK * EMBED_DIM)],
                sem_w.at[b])
        return carry

    lax.fori_loop(0, n_chunks // 2, pair, 0)

    for b in range(2):
        e0 = base + (n_chunks - 2 + b) * CHUNK
        pltpu.make_async_copy(
            rows_v.at[pl.ds(b * CHUNK * EMBED_DIM, CHUNK * EMBED_DIM)],
            out1.at[pl.ds(e0 * EMBED_DIM, CHUNK * EMBED_DIM)],
            sem_w.at[b]).wait()


@jax.jit
def kernel(z, table):
    B, S = z.shape
    n = B * S
    z_flat = z.reshape(n).astype(jnp.int32)
    table = table.at[0].set(jnp.zeros((table.shape[1],), table.dtype))
    table_flat = table.reshape(table.shape[0] * table.shape[1])

    mesh = plsc.VectorSubcoreMesh(core_axis_name="c", subcore_axis_name="s")
    out1 = pl.kernel(
        _emb_body,
        mesh=mesh,
        out_type=jax.ShapeDtypeStruct((n * EMBED_DIM,), jnp.float32),
        scratch_types=[
            pltpu.VMEM((100 * EMBED_DIM,), jnp.float32),
            pltpu.VMEM((2 * CHUNK,), jnp.int32),
            pltpu.VMEM((2 * CHUNK * EMBED_DIM,), jnp.float32),
            pltpu.SemaphoreType.DMA((2,)),
            pltpu.SemaphoreType.DMA((2,)),
        ],
        compiler_params=pltpu.CompilerParams(use_tc_tiling_on_sc=True,
                                             needs_layout_passes=False),
    )(z_flat, table_flat)
    return out1.reshape(B, S, EMBED_DIM)
